# baseline (speedup 1.0000x reference)
import functools
import math

import jax
import jax.numpy as jnp
from jax import lax
from jax.experimental import pallas as pl
from jax.experimental.pallas import tpu as pltpu
from jax.experimental.pallas import tpu_sc as plsc

_D = 128
_S = 200
_B = 1024
_N = _B * _S

_info = plsc.get_sparse_core_info()
_NC = _info.num_cores
_NS = _info.num_subcores
_NW = _NC * _NS
_NSLICE = 4
_BSL = _B // _NSLICE
_NSL = _N // _NSLICE
_PER_W = _NSL // _NW
_CHUNK = 64
_CHUNKS = _PER_W // _CHUNK


def _sc_gather(tok3d, table):
    mesh = plsc.VectorSubcoreMesh(core_axis_name="c", subcore_axis_name="s")

    @functools.partial(
        pl.kernel,
        out_type=jax.ShapeDtypeStruct((_NSL, _D), jnp.float32),
        mesh=mesh,
        scratch_types=[
            pltpu.VMEM((_CHUNKS, _CHUNK), jnp.int32),
            pltpu.VMEM((_CHUNK, _D), jnp.float32),
            pltpu.VMEM((_CHUNK, _D), jnp.float32),
            pltpu.SemaphoreType.DMA,
            pltpu.SemaphoreType.DMA,
        ],
    )
    def k(tok_hbm, table_hbm, out_hbm, idx_v, rows0, rows1, sem0, sem1):
        wid = lax.axis_index("s") * _NC + lax.axis_index("c")
        base = wid * _PER_W
        pltpu.sync_copy(tok_hbm.at[wid], idx_v)

        pltpu.async_copy(table_hbm.at[idx_v.at[0]], rows0, sem0)

        def step(c, carry):
            def turn(mine, other, msem, osem):
                @pl.when(c + 1 < _CHUNKS)
                def _():
                    pltpu.async_copy(table_hbm.at[idx_v.at[c + 1]], other, osem)

                pltpu.make_async_copy(
                    table_hbm.at[idx_v.at[c]], mine, msem
                ).wait()
                pltpu.sync_copy(
                    mine, out_hbm.at[pl.ds(base + c * _CHUNK, _CHUNK)]
                )

            @pl.when(lax.rem(c, 2) == 0)
            def _():
                turn(rows0, rows1, sem0, sem1)

            @pl.when(lax.rem(c, 2) == 1)
            def _():
                turn(rows1, rows0, sem1, sem0)

            return carry

        lax.fori_loop(0, _CHUNKS, step, 0)

    return k(tok3d, table)


_BB = 8


def _tc_body(g_ref, pos_ref, w_ref, acc_ref, o_ref):
    del acc_ref
    scale = math.sqrt(_D)
    x = g_ref[...] + pos_ref[...][None]
    ws = w_ref[...] * scale
    xf = x.reshape(_BB * _S, _D)
    y = lax.dot_general(
        xf, ws, (((1,), (1,)), ((), ())), preferred_element_type=jnp.float32
    )
    o_ref[...] = y.reshape(_BB, _S, _D)


def _tc_project(sl, g3d, pos, w, acc):
    off = sl * (_BSL // _BB)
    return pl.pallas_call(
        _tc_body,
        grid=(_BSL // _BB,),
        in_specs=[
            pl.BlockSpec((_BB, _S, _D), lambda i: (i, 0, 0)),
            pl.BlockSpec((_S, _D), lambda i: (0, 0)),
            pl.BlockSpec((_D, _D), lambda i: (0, 0)),
            pl.BlockSpec(memory_space=pl.ANY),
        ],
        out_specs=pl.BlockSpec((_BB, _S, _D), lambda i: (off + i, 0, 0)),
        out_shape=jax.ShapeDtypeStruct((_B, _S, _D), jnp.float32),
        input_output_aliases={3: 0},
    )(g3d, pos, w, acc)


@jax.jit
def kernel(tokens, emb_weight, pos_weight, proj_weight):
    tok = tokens.astype(jnp.int32).reshape(_NSLICE, _NW, _CHUNKS, _CHUNK)
    gathered = [_sc_gather(tok[sl], emb_weight) for sl in range(_NSLICE)]
    acc = jnp.zeros((_B, _S, _D), jnp.float32)
    for sl in range(_NSLICE):
        g3d = gathered[sl].reshape(_BSL, _S, _D)
        acc = _tc_project(sl, g3d, pos_weight, proj_weight, acc)
    return acc


# speedup vs baseline: 1.5487x; 1.5487x over previous
import functools
import math

import jax
import jax.numpy as jnp
from jax import lax
from jax.experimental import pallas as pl
from jax.experimental.pallas import tpu as pltpu
from jax.experimental.pallas import tpu_sc as plsc

_D = 128
_S = 200
_B = 1024
_N = _B * _S

_info = plsc.get_sparse_core_info()
_NC = _info.num_cores
_NS = _info.num_subcores
_NW = _NC * _NS
_PER_W = _N // _NW
_CHUNK = 128
_CHUNKS = _PER_W // _CHUNK
_NBUF = 4


def _sc_gather(tok3d, table):
    mesh = plsc.VectorSubcoreMesh(core_axis_name="c", subcore_axis_name="s")

    @functools.partial(
        pl.kernel,
        out_type=jax.ShapeDtypeStruct((_N, _D), jnp.float32),
        mesh=mesh,
        scratch_types=[
            pltpu.VMEM((_CHUNKS, _CHUNK), jnp.int32),
            *([pltpu.VMEM((_CHUNK, _D), jnp.float32)] * _NBUF),
            *([pltpu.SemaphoreType.DMA] * _NBUF),
            *([pltpu.SemaphoreType.DMA] * _NBUF),
        ],
    )
    def k(tok_hbm, table_hbm, out_hbm, idx_v, *bufsems):
        rows = bufsems[:_NBUF]
        gsem = bufsems[_NBUF : 2 * _NBUF]
        ssem = bufsems[2 * _NBUF :]
        wid = lax.axis_index("s") * _NC + lax.axis_index("c")
        base = wid * _PER_W
        pltpu.sync_copy(tok_hbm.at[wid], idx_v)

        for j in range(_NBUF - 1):
            pltpu.async_copy(table_hbm.at[idx_v.at[j]], rows[j], gsem[j])

        def turn(c, j):
            b3 = (j + _NBUF - 1) % _NBUF
            pltpu.make_async_copy(
                table_hbm.at[idx_v.at[c]], rows[j], gsem[j]
            ).wait()
            pltpu.async_copy(
                rows[j], out_hbm.at[pl.ds(base + c * _CHUNK, _CHUNK)], ssem[j]
            )

            @pl.when(c + _NBUF - 1 < _CHUNKS)
            def _():
                @pl.when(c >= 1)
                def _():
                    pltpu.make_async_copy(
                        rows[b3],
                        out_hbm.at[pl.ds(base + (c - 1) * _CHUNK, _CHUNK)],
                        ssem[b3],
                    ).wait()

                pltpu.async_copy(
                    table_hbm.at[idx_v.at[c + _NBUF - 1]], rows[b3], gsem[b3]
                )

        def step(c, carry):
            for j in range(_NBUF):
                @pl.when(lax.rem(c, _NBUF) == j)
                def _(c=c, j=j):
                    turn(c, j)
            return carry

        lax.fori_loop(0, _CHUNKS, step, 0)

        for j in range(_NBUF):
            pltpu.make_async_copy(
                rows[j], out_hbm.at[pl.ds(0, _CHUNK)], ssem[j]
            ).wait()

    return k(tok3d, table)


_BB = 32


def _tc_body(g_ref, pos_ref, w_ref, o_ref):
    scale = math.sqrt(_D)
    x = g_ref[...] + pos_ref[...][None]
    ws = w_ref[...] * scale
    xf = x.reshape(_BB * _S, _D)
    y = lax.dot_general(
        xf, ws, (((1,), (1,)), ((), ())), preferred_element_type=jnp.float32
    )
    o_ref[...] = y.reshape(_BB, _S, _D)


def _tc_project(g3d, pos, w):
    return pl.pallas_call(
        _tc_body,
        grid=(_B // _BB,),
        in_specs=[
            pl.BlockSpec((_BB, _S, _D), lambda i: (i, 0, 0)),
            pl.BlockSpec((_S, _D), lambda i: (0, 0)),
            pl.BlockSpec((_D, _D), lambda i: (0, 0)),
        ],
        out_specs=pl.BlockSpec((_BB, _S, _D), lambda i: (i, 0, 0)),
        out_shape=jax.ShapeDtypeStruct((_B, _S, _D), jnp.float32),
    )(g3d, pos, w)


@jax.jit
def kernel(tokens, emb_weight, pos_weight, proj_weight):
    tok3d = tokens.astype(jnp.int32).reshape(_NW, _CHUNKS, _CHUNK)
    gathered = _sc_gather(tok3d, emb_weight)
    g3d = gathered.reshape(_B, _S, _D)
    return _tc_project(g3d, pos_weight, proj_weight)


# speedup vs baseline: 1.5597x; 1.0071x over previous
import functools
import math

import jax
import jax.numpy as jnp
from jax import lax
from jax.experimental import pallas as pl
from jax.experimental.pallas import tpu as pltpu
from jax.experimental.pallas import tpu_sc as plsc

_D = 128
_S = 200
_B = 1024
_N = _B * _S

_info = plsc.get_sparse_core_info()
_NC = _info.num_cores
_NS = _info.num_subcores
_NW = _NC * _NS
_NSLICE = 2
_BSL = _B // _NSLICE
_NSL = _N // _NSLICE
_PER_W = _NSL // _NW
_CHUNK = 128
_CHUNKS = _PER_W // _CHUNK
_NBUF = 4


def _sc_gather(tok3d, table):
    mesh = plsc.VectorSubcoreMesh(core_axis_name="c", subcore_axis_name="s")

    @functools.partial(
        pl.kernel,
        out_type=jax.ShapeDtypeStruct((_NSL, _D), jnp.float32),
        mesh=mesh,
        scratch_types=[
            pltpu.VMEM((_CHUNKS, _CHUNK), jnp.int32),
            *([pltpu.VMEM((_CHUNK, _D), jnp.float32)] * _NBUF),
            *([pltpu.SemaphoreType.DMA] * _NBUF),
            *([pltpu.SemaphoreType.DMA] * _NBUF),
        ],
    )
    def k(tok_hbm, table_hbm, out_hbm, idx_v, *bufsems):
        rows = bufsems[:_NBUF]
        gsem = bufsems[_NBUF : 2 * _NBUF]
        ssem = bufsems[2 * _NBUF :]
        wid = lax.axis_index("s") * _NC + lax.axis_index("c")
        base = wid * _PER_W
        pltpu.sync_copy(tok_hbm.at[wid], idx_v)

        for j in range(_NBUF - 1):
            pltpu.async_copy(table_hbm.at[idx_v.at[j]], rows[j], gsem[j])

        def turn(c, j):
            b3 = (j + _NBUF - 1) % _NBUF
            pltpu.make_async_copy(
                table_hbm.at[idx_v.at[c]], rows[j], gsem[j]
            ).wait()
            pltpu.async_copy(
                rows[j], out_hbm.at[pl.ds(base + c * _CHUNK, _CHUNK)], ssem[j]
            )

            @pl.when(c + _NBUF - 1 < _CHUNKS)
            def _():
                @pl.when(c >= 1)
                def _():
                    pltpu.make_async_copy(
                        rows[b3],
                        out_hbm.at[pl.ds(base + (c - 1) * _CHUNK, _CHUNK)],
                        ssem[b3],
                    ).wait()

                pltpu.async_copy(
                    table_hbm.at[idx_v.at[c + _NBUF - 1]], rows[b3], gsem[b3]
                )

        def step(c, carry):
            for j in range(_NBUF):
                @pl.when(lax.rem(c, _NBUF) == j)
                def _(c=c, j=j):
                    turn(c, j)
            return carry

        lax.fori_loop(0, _CHUNKS, step, 0)

        for j in range(_NBUF):
            pltpu.make_async_copy(
                rows[j], out_hbm.at[pl.ds(0, _CHUNK)], ssem[j]
            ).wait()

    return k(tok3d, table)


_BB = 32


def _tc_body_first(g_ref, pos_ref, w_ref, o_ref):
    scale = math.sqrt(_D)
    x = g_ref[...] + pos_ref[...][None]
    ws = w_ref[...] * scale
    xf = x.reshape(_BB * _S, _D)
    y = lax.dot_general(
        xf, ws, (((1,), (1,)), ((), ())), preferred_element_type=jnp.float32
    )
    o_ref[...] = y.reshape(_BB, _S, _D)


def _tc_body(g_ref, pos_ref, w_ref, acc_ref, o_ref):
    del acc_ref
    _tc_body_first(g_ref, pos_ref, w_ref, o_ref)


def _tc_project(sl, g3d, pos, w, acc):
    off = sl * (_BSL // _BB)
    specs = [
        pl.BlockSpec((_BB, _S, _D), lambda i: (i, 0, 0)),
        pl.BlockSpec((_S, _D), lambda i: (0, 0)),
        pl.BlockSpec((_D, _D), lambda i: (0, 0)),
    ]
    args = (g3d, pos, w)
    body = _tc_body_first
    aliases = {}
    if acc is not None:
        specs.append(pl.BlockSpec(memory_space=pl.ANY))
        args = args + (acc,)
        body = _tc_body
        aliases = {3: 0}
    return pl.pallas_call(
        body,
        grid=(_BSL // _BB,),
        in_specs=specs,
        out_specs=pl.BlockSpec((_BB, _S, _D), lambda i: (off + i, 0, 0)),
        out_shape=jax.ShapeDtypeStruct((_B, _S, _D), jnp.float32),
        input_output_aliases=aliases,
    )(*args)


@jax.jit
def kernel(tokens, emb_weight, pos_weight, proj_weight):
    tok = tokens.astype(jnp.int32).reshape(_NSLICE, _NW, _CHUNKS, _CHUNK)
    gathered = [_sc_gather(tok[sl], emb_weight) for sl in range(_NSLICE)]
    acc = None
    for sl in range(_NSLICE):
        g3d = gathered[sl].reshape(_BSL, _S, _D)
        acc = _tc_project(sl, g3d, pos_weight, proj_weight, acc)
    return acc
